# baseline (speedup 1.0000x reference)
import math
import numpy as np
import jax
import jax.numpy as jnp
from jax.experimental import pallas as pl
from jax.experimental.pallas import tpu as pltpu

_DEFAULT = jax.lax.Precision.DEFAULT


def _jpeg_quant_tables():
    y_table = np.array(
        [[16, 11, 10, 16, 24, 40, 51, 61],
         [12, 12, 14, 19, 26, 58, 60, 55],
         [14, 13, 16, 24, 40, 57, 69, 56],
         [14, 17, 22, 29, 51, 87, 80, 62],
         [18, 22, 37, 56, 68, 109, 103, 77],
         [24, 35, 55, 64, 81, 104, 113, 92],
         [49, 64, 78, 87, 103, 121, 120, 101],
         [72, 92, 95, 98, 112, 100, 103, 99]], dtype=np.float32).T
    c_table = np.full((8, 8), 99.0, dtype=np.float32)
    c_table[:4, :4] = np.array([[17, 18, 24, 47],
                                [18, 21, 26, 66],
                                [24, 26, 56, 99],
                                [47, 66, 99, 99]], dtype=np.float32).T
    return y_table, c_table


def _idct_basis():
    alpha = np.array([1.0 / np.sqrt(2.0)] + [1.0] * 7, dtype=np.float32)
    alpha2 = np.outer(alpha, alpha).astype(np.float32)
    basis = np.zeros((8, 8, 8, 8), dtype=np.float32)
    for x in range(8):
        for y in range(8):
            for u in range(8):
                for v in range(8):
                    basis[x, y, u, v] = (math.cos((2 * u + 1) * x * math.pi / 16) *
                                         math.cos((2 * v + 1) * y * math.pi / 16))
    return (alpha2[:, :, None, None] * basis).reshape(64, 64).astype(np.float32)


def _pack_basis(scaled, pack, dup):
    k = 64 * pack
    g = np.zeros((8, k, 128), np.float32)
    for s1 in range(8):
        cols = np.repeat(scaled[:, s1 * 8:(s1 + 1) * 8], dup, axis=1)
        w = 8 * dup
        for j in range(pack):
            g[s1, j * 64:(j + 1) * 64, j * w:(j + 1) * w] = cols
    return g.transpose(1, 0, 2).reshape(k, 8 * 128)


def _split_hi_lo(g):
    hi = np.asarray(g.astype(jnp.bfloat16), np.float32)
    lo = np.asarray((g - hi).astype(jnp.bfloat16), np.float32)
    return hi, lo


def _fused_kernel(th, w, tny, tnc):
    tbh = th // 8
    cbh = th // 16
    nxt = w // 128
    ty, tc = tbh * nxt, cbh * nxt
    py, pc = ty + 8, tc + 8

    def body(q_ref, yt_ref, cbt_ref, crt_ref, r_ref, gyh_ref,
             gch_ref, out_ref, dsc_ref, psc_ref,
             ysc_ref, cbsc_ref, crsc_ref):
        b = pl.program_id(0)
        s = q_ref[b] * 0.25
        r = r_ref[...]

        def packed_lhs(t_ref, n, npack):
            dup = jax.lax.dot_general(
                t_ref[0], r, (((0,), (0,)), ((), ())),
                preferred_element_type=jnp.float32, precision=_DEFAULT)
            dsc_ref[0:n, :] = dup
            ev = dsc_ref[pl.ds(0, n // 2, 2)]
            od = dsc_ref[pl.ds(1, n // 2, 2)]
            lane = jax.lax.broadcasted_iota(jnp.int32, (n // 2, 128), 1)
            psc_ref[0:n // 2, :] = jnp.where(lane < 64, ev, od)
            half = npack // 2
            return jnp.concatenate(
                [psc_ref[pl.ds(jp, n // npack, half)] for jp in range(half)],
                axis=1)

        ylhs = packed_lhs(yt_ref, tny, 16)
        ymm = jnp.dot(ylhs, gyh_ref[...], preferred_element_type=jnp.float32,
                      precision=_DEFAULT)
        ymm = ymm * s + 128.0
        for s1 in range(8):
            ysc_ref[s1 * py:s1 * py + ty, :] = ymm[:, s1 * 128:(s1 + 1) * 128]

        cblhs = packed_lhs(cbt_ref, tnc, 8)
        crlhs = packed_lhs(crt_ref, tnc, 8)
        cbf = jnp.concatenate([cblhs, crlhs], axis=0)
        cmm = jnp.dot(cbf, gch_ref[...], preferred_element_type=jnp.float32,
                      precision=_DEFAULT)
        cmm = cmm * s
        for s1 in range(8):
            cbp = cmm[:tc, s1 * 128:(s1 + 1) * 128]
            crp = cmm[tc:, s1 * 128:(s1 + 1) * 128]
            for e in (0, 1):
                yp = (2 * s1 + e) * pc
                cbsc_ref[yp:yp + tc, :] = cbp
                crsc_ref[yp:yp + tc, :] = crp

        inv255 = 1.0 / 255.0
        for xt in range(nxt):
            yb = jnp.concatenate(
                [ysc_ref[pl.ds(br * nxt + xt, 8, py)] for br in range(tbh)],
                axis=0)
            cbb = jnp.concatenate(
                [cbsc_ref[pl.ds((g % 2) * 8 * pc + (g // 2) * nxt + xt, 8, pc)]
                 for g in range(th // 8)], axis=0)
            crb = jnp.concatenate(
                [crsc_ref[pl.ds((g % 2) * 8 * pc + (g // 2) * nxt + xt, 8, pc)]
                 for g in range(th // 8)], axis=0)
            r_ = yb + 1.402 * crb
            g_ = yb - 0.344136 * cbb - 0.714136 * crb
            bl = yb + 1.772 * cbb
            cs = slice(xt * 128, (xt + 1) * 128)
            out_ref[0, 0, :, cs] = jnp.clip(r_, 0.0, 255.0) * inv255
            out_ref[0, 1, :, cs] = jnp.clip(g_, 0.0, 255.0) * inv255
            out_ref[0, 2, :, cs] = jnp.clip(bl, 0.0, 255.0) * inv255

    return body


def _diffjpeg(y, cb, cr, quantization, height, width, th):
    B = y.shape[0]
    ny, nc = y.shape[1], cb.shape[1]
    assert ny == (height // 8) * (width // 8) and nc == (height // 16) * (width // 16)
    assert width % 128 == 0

    y_t, c_t = _jpeg_quant_tables()
    b64 = _idct_basis()
    gy = _pack_basis(y_t.reshape(64, 1) * b64, pack=16, dup=1)
    gc = _pack_basis(c_t.reshape(64, 1) * b64, pack=8, dup=2)
    gyh = np.asarray(gy.astype(jnp.bfloat16), np.float32)
    gch = np.asarray(gc.astype(jnp.bfloat16), np.float32)

    yt = jnp.transpose(y, (0, 2, 3, 1)).reshape(B, 64, ny)
    cbt = jnp.transpose(cb, (0, 2, 3, 1)).reshape(B, 64, nc)
    crt = jnp.transpose(cr, (0, 2, 3, 1)).reshape(B, 64, nc)
    rdup = np.concatenate([np.eye(64, dtype=np.float32)] * 2, axis=1)

    tbh, cbh, nxt = th // 8, th // 16, width // 128
    ty, tc = tbh * nxt, cbh * nxt
    tny, tnc = (th // 8) * (width // 8), (th // 16) * (width // 16)

    return pl.pallas_call(
        _fused_kernel(th, width, tny, tnc),
        out_shape=jax.ShapeDtypeStruct((B, 3, height, width), jnp.float32),
        grid=(B, height // th),
        in_specs=[
            pl.BlockSpec(memory_space=pltpu.SMEM),
            pl.BlockSpec((1, 64, tny), lambda b, t: (b, 0, t)),
            pl.BlockSpec((1, 64, tnc), lambda b, t: (b, 0, t)),
            pl.BlockSpec((1, 64, tnc), lambda b, t: (b, 0, t)),
            pl.BlockSpec((64, 128), lambda b, t: (0, 0)),
            pl.BlockSpec((1024, 1024), lambda b, t: (0, 0)),
            pl.BlockSpec((512, 1024), lambda b, t: (0, 0)),
        ],
        out_specs=pl.BlockSpec((1, 3, th, width), lambda b, t: (b, 0, t, 0)),
        scratch_shapes=[
            pltpu.VMEM((tny, 128), jnp.float32),
            pltpu.VMEM((tny // 2, 128), jnp.float32),
            pltpu.VMEM((8 * (ty + 8), 128), jnp.float32),
            pltpu.VMEM((16 * (tc + 8), 128), jnp.float32),
            pltpu.VMEM((16 * (tc + 8), 128), jnp.float32),
        ],
        compiler_params=pltpu.CompilerParams(
            dimension_semantics=("parallel", "parallel"),
            vmem_limit_bytes=100 * 1024 * 1024),
    )(jnp.asarray(quantization, jnp.float32), yt, cbt, crt, jnp.asarray(rdup),
      jnp.asarray(gyh), jnp.asarray(gch))


def kernel(y, cb, cr, quantization):
    return _diffjpeg(y, cb, cr, quantization, 512, 512, 256)


# speedup vs baseline: 1.1289x; 1.1289x over previous
import math
import numpy as np
import jax
import jax.numpy as jnp
from jax.experimental import pallas as pl
from jax.experimental.pallas import tpu as pltpu

_DEFAULT = jax.lax.Precision.DEFAULT


def _jpeg_quant_tables():
    y_table = np.array(
        [[16, 11, 10, 16, 24, 40, 51, 61],
         [12, 12, 14, 19, 26, 58, 60, 55],
         [14, 13, 16, 24, 40, 57, 69, 56],
         [14, 17, 22, 29, 51, 87, 80, 62],
         [18, 22, 37, 56, 68, 109, 103, 77],
         [24, 35, 55, 64, 81, 104, 113, 92],
         [49, 64, 78, 87, 103, 121, 120, 101],
         [72, 92, 95, 98, 112, 100, 103, 99]], dtype=np.float32).T
    c_table = np.full((8, 8), 99.0, dtype=np.float32)
    c_table[:4, :4] = np.array([[17, 18, 24, 47],
                                [18, 21, 26, 66],
                                [24, 26, 56, 99],
                                [47, 66, 99, 99]], dtype=np.float32).T
    return y_table, c_table


def _idct_basis():
    alpha = np.array([1.0 / np.sqrt(2.0)] + [1.0] * 7, dtype=np.float32)
    alpha2 = np.outer(alpha, alpha).astype(np.float32)
    basis = np.zeros((8, 8, 8, 8), dtype=np.float32)
    for x in range(8):
        for y in range(8):
            for u in range(8):
                for v in range(8):
                    basis[x, y, u, v] = (math.cos((2 * u + 1) * x * math.pi / 16) *
                                         math.cos((2 * v + 1) * y * math.pi / 16))
    return (alpha2[:, :, None, None] * basis).reshape(64, 64).astype(np.float32)


def _pack_basis(scaled, pack, dup):
    k = 64 * pack
    g = np.zeros((8, k, 128), np.float32)
    for s1 in range(8):
        cols = np.repeat(scaled[:, s1 * 8:(s1 + 1) * 8], dup, axis=1)
        w = 8 * dup
        for j in range(pack):
            g[s1, j * 64:(j + 1) * 64, j * w:(j + 1) * w] = cols
    return g.transpose(1, 0, 2).reshape(k, 8 * 128)


def _split_hi_lo(g):
    hi = np.asarray(g.astype(jnp.bfloat16), np.float32)
    lo = np.asarray((g - hi).astype(jnp.bfloat16), np.float32)
    return hi, lo


def _fused_kernel(th, w, tny, tnc):
    tbh = th // 8
    cbh = th // 16
    nxt = w // 128
    ty, tc = tbh * nxt, cbh * nxt
    py, pc = ty + 8, tc + 8

    def body(q_ref, yt_ref, cbt_ref, crt_ref, r_ref, gyh_ref,
             gch_ref, out_ref, dsc_ref, psc_ref,
             ysc_ref, cbsc_ref, crsc_ref):
        b = pl.program_id(0)
        s = q_ref[b] * 0.25
        r = r_ref[...]

        def packed_lhs(t_ref, n, npack):
            dup = jax.lax.dot_general(
                t_ref[0], r, (((0,), (0,)), ((), ())),
                preferred_element_type=jnp.float32, precision=_DEFAULT)
            dsc_ref[0:n, :] = dup
            ev = dsc_ref[pl.ds(0, n // 2, 2)]
            od = dsc_ref[pl.ds(1, n // 2, 2)]
            lane = jax.lax.broadcasted_iota(jnp.int32, (n // 2, 128), 1)
            psc_ref[0:n // 2, :] = jnp.where(lane < 64, ev, od)
            half = npack // 2
            return jnp.concatenate(
                [psc_ref[pl.ds(jp, n // npack, half)] for jp in range(half)],
                axis=1)

        ylhs = packed_lhs(yt_ref, tny, 16)
        ymm = jnp.dot(ylhs, gyh_ref[...], preferred_element_type=jnp.float32,
                      precision=_DEFAULT)
        ymm = ymm * s + 128.0
        for s1 in range(8):
            ysc_ref[s1 * py:s1 * py + ty, :] = ymm[:, s1 * 128:(s1 + 1) * 128]

        cblhs = packed_lhs(cbt_ref, tnc, 8)
        crlhs = packed_lhs(crt_ref, tnc, 8)
        cbf = jnp.concatenate([cblhs, crlhs], axis=0)
        cmm = jnp.dot(cbf, gch_ref[...], preferred_element_type=jnp.float32,
                      precision=_DEFAULT)
        cmm = cmm * s
        for s1 in range(8):
            cbp = cmm[:tc, s1 * 128:(s1 + 1) * 128]
            crp = cmm[tc:, s1 * 128:(s1 + 1) * 128]
            for e in (0, 1):
                yp = (2 * s1 + e) * pc
                cbsc_ref[yp:yp + tc, :] = cbp
                crsc_ref[yp:yp + tc, :] = crp

        inv255 = 1.0 / 255.0
        for xt in range(nxt):
            yb = jnp.concatenate(
                [ysc_ref[pl.ds(br * nxt + xt, 8, py)] for br in range(tbh)],
                axis=0)
            cbb = jnp.concatenate(
                [cbsc_ref[pl.ds((g % 2) * 8 * pc + (g // 2) * nxt + xt, 8, pc)]
                 for g in range(th // 8)], axis=0)
            crb = jnp.concatenate(
                [crsc_ref[pl.ds((g % 2) * 8 * pc + (g // 2) * nxt + xt, 8, pc)]
                 for g in range(th // 8)], axis=0)
            r_ = yb + 1.402 * crb
            g_ = yb - 0.344136 * cbb - 0.714136 * crb
            bl = yb + 1.772 * cbb
            cs = slice(xt * 128, (xt + 1) * 128)
            out_ref[0, 0, :, cs] = jnp.clip(r_, 0.0, 255.0) * inv255
            out_ref[0, 1, :, cs] = jnp.clip(g_, 0.0, 255.0) * inv255
            out_ref[0, 2, :, cs] = jnp.clip(bl, 0.0, 255.0) * inv255

    return body


def _diffjpeg(y, cb, cr, quantization, height, width, th):
    B = y.shape[0]
    ny, nc = y.shape[1], cb.shape[1]
    assert ny == (height // 8) * (width // 8) and nc == (height // 16) * (width // 16)
    assert width % 128 == 0

    y_t, c_t = _jpeg_quant_tables()
    b64 = _idct_basis()
    gy = _pack_basis(y_t.reshape(64, 1) * b64, pack=16, dup=1)
    gc = _pack_basis(c_t.reshape(64, 1) * b64, pack=8, dup=2)
    gyh = np.asarray(gy.astype(jnp.bfloat16), np.float32)
    gch = np.asarray(gc.astype(jnp.bfloat16), np.float32)

    yt = jnp.transpose(y, (0, 2, 3, 1)).reshape(B, 64, ny)
    cbt = jnp.transpose(cb, (0, 2, 3, 1)).reshape(B, 64, nc)
    crt = jnp.transpose(cr, (0, 2, 3, 1)).reshape(B, 64, nc)
    rdup = np.concatenate([np.eye(64, dtype=np.float32)] * 2, axis=1)

    tbh, cbh, nxt = th // 8, th // 16, width // 128
    ty, tc = tbh * nxt, cbh * nxt
    tny, tnc = (th // 8) * (width // 8), (th // 16) * (width // 16)

    return pl.pallas_call(
        _fused_kernel(th, width, tny, tnc),
        out_shape=jax.ShapeDtypeStruct((B, 3, height, width), jnp.float32),
        grid=(B, height // th),
        in_specs=[
            pl.BlockSpec(memory_space=pltpu.SMEM),
            pl.BlockSpec((1, 64, tny), lambda b, t: (b, 0, t)),
            pl.BlockSpec((1, 64, tnc), lambda b, t: (b, 0, t)),
            pl.BlockSpec((1, 64, tnc), lambda b, t: (b, 0, t)),
            pl.BlockSpec((64, 128), lambda b, t: (0, 0)),
            pl.BlockSpec((1024, 1024), lambda b, t: (0, 0)),
            pl.BlockSpec((512, 1024), lambda b, t: (0, 0)),
        ],
        out_specs=pl.BlockSpec((1, 3, th, width), lambda b, t: (b, 0, t, 0)),
        scratch_shapes=[
            pltpu.VMEM((tny, 128), jnp.float32),
            pltpu.VMEM((tny // 2, 128), jnp.float32),
            pltpu.VMEM((8 * (ty + 8), 128), jnp.float32),
            pltpu.VMEM((16 * (tc + 8), 128), jnp.float32),
            pltpu.VMEM((16 * (tc + 8), 128), jnp.float32),
        ],
        compiler_params=pltpu.CompilerParams(
            dimension_semantics=("parallel", "parallel"),
            vmem_limit_bytes=100 * 1024 * 1024),
    )(jnp.asarray(quantization, jnp.float32), yt, cbt, crt, jnp.asarray(rdup),
      jnp.asarray(gyh), jnp.asarray(gch))


def kernel(y, cb, cr, quantization):
    return _diffjpeg(y, cb, cr, quantization, 512, 512, 512)


# speedup vs baseline: 1.1290x; 1.0001x over previous
import math
import numpy as np
import jax
import jax.numpy as jnp
from jax.experimental import pallas as pl
from jax.experimental.pallas import tpu as pltpu

_DEFAULT = jax.lax.Precision.DEFAULT


def _jpeg_quant_tables():
    y_table = np.array(
        [[16, 11, 10, 16, 24, 40, 51, 61],
         [12, 12, 14, 19, 26, 58, 60, 55],
         [14, 13, 16, 24, 40, 57, 69, 56],
         [14, 17, 22, 29, 51, 87, 80, 62],
         [18, 22, 37, 56, 68, 109, 103, 77],
         [24, 35, 55, 64, 81, 104, 113, 92],
         [49, 64, 78, 87, 103, 121, 120, 101],
         [72, 92, 95, 98, 112, 100, 103, 99]], dtype=np.float32).T
    c_table = np.full((8, 8), 99.0, dtype=np.float32)
    c_table[:4, :4] = np.array([[17, 18, 24, 47],
                                [18, 21, 26, 66],
                                [24, 26, 56, 99],
                                [47, 66, 99, 99]], dtype=np.float32).T
    return y_table, c_table


def _idct_basis():
    alpha = np.array([1.0 / np.sqrt(2.0)] + [1.0] * 7, dtype=np.float32)
    alpha2 = np.outer(alpha, alpha).astype(np.float32)
    basis = np.zeros((8, 8, 8, 8), dtype=np.float32)
    for x in range(8):
        for y in range(8):
            for u in range(8):
                for v in range(8):
                    basis[x, y, u, v] = (math.cos((2 * u + 1) * x * math.pi / 16) *
                                         math.cos((2 * v + 1) * y * math.pi / 16))
    return (alpha2[:, :, None, None] * basis).reshape(64, 64).astype(np.float32)


def _pack_basis(scaled, pack, dup):
    k = 64 * pack
    g = np.zeros((8, k, 128), np.float32)
    for s1 in range(8):
        cols = np.repeat(scaled[:, s1 * 8:(s1 + 1) * 8], dup, axis=1)
        w = 8 * dup
        for j in range(pack):
            g[s1, j * 64:(j + 1) * 64, j * w:(j + 1) * w] = cols
    return g.transpose(1, 0, 2).reshape(k, 8 * 128)


def _fused_kernel(th, w, tny, tnc):
    tbh = th // 8
    cbh = th // 16
    nxt = w // 128
    ty, tc = tbh * nxt, cbh * nxt
    py, pc = ty + 8, tc + 8

    def body(q_ref, yt_ref, cbt_ref, crt_ref, r_ref, gyh_ref,
             gch_ref, out_ref, dsc_ref, psc_ref,
             ysc_ref, cbsc_ref, crsc_ref):
        b = pl.program_id(0)
        s = q_ref[b] * 0.25
        r = r_ref[...]

        def packed_lhs(t_ref, n, npack):
            dup = jax.lax.dot_general(
                t_ref[0], r, (((0,), (0,)), ((), ())),
                preferred_element_type=jnp.float32, precision=_DEFAULT)
            dsc_ref[0:n, :] = dup
            ev = dsc_ref[pl.ds(0, n // 2, 2)]
            od = dsc_ref[pl.ds(1, n // 2, 2)]
            lane = jax.lax.broadcasted_iota(jnp.int32, (n // 2, 128), 1)
            psc_ref[0:n // 2, :] = jnp.where(lane < 64, ev, od)
            half = npack // 2
            return jnp.concatenate(
                [psc_ref[pl.ds(jp, n // npack, half)] for jp in range(half)],
                axis=1)

        ylhs = packed_lhs(yt_ref, tny, 16)
        ymm = jnp.dot(ylhs, gyh_ref[...], preferred_element_type=jnp.float32,
                      precision=_DEFAULT)
        ymm = ymm * s + 128.0
        for s1 in range(8):
            ysc_ref[s1 * py:s1 * py + ty, :] = ymm[:, s1 * 128:(s1 + 1) * 128]

        cblhs = packed_lhs(cbt_ref, tnc, 8)
        crlhs = packed_lhs(crt_ref, tnc, 8)
        cbf = jnp.concatenate([cblhs, crlhs], axis=0)
        cmm = jnp.dot(cbf, gch_ref[...], preferred_element_type=jnp.float32,
                      precision=_DEFAULT)
        cmm = cmm * s
        for s1 in range(8):
            cbp = cmm[:tc, s1 * 128:(s1 + 1) * 128]
            crp = cmm[tc:, s1 * 128:(s1 + 1) * 128]
            for e in (0, 1):
                yp = (2 * s1 + e) * pc
                cbsc_ref[yp:yp + tc, :] = cbp
                crsc_ref[yp:yp + tc, :] = crp

        inv255 = 1.0 / 255.0
        for xt in range(nxt):
            yb = jnp.concatenate(
                [ysc_ref[pl.ds(br * nxt + xt, 8, py)] for br in range(tbh)],
                axis=0)
            cbb = jnp.concatenate(
                [cbsc_ref[pl.ds((g % 2) * 8 * pc + (g // 2) * nxt + xt, 8, pc)]
                 for g in range(th // 8)], axis=0)
            crb = jnp.concatenate(
                [crsc_ref[pl.ds((g % 2) * 8 * pc + (g // 2) * nxt + xt, 8, pc)]
                 for g in range(th // 8)], axis=0)
            r_ = yb + 1.402 * crb
            g_ = yb - 0.344136 * cbb - 0.714136 * crb
            bl = yb + 1.772 * cbb
            cs = slice(xt * 128, (xt + 1) * 128)
            out_ref[0, 0, :, cs] = jnp.clip(r_, 0.0, 255.0) * inv255
            out_ref[0, 1, :, cs] = jnp.clip(g_, 0.0, 255.0) * inv255
            out_ref[0, 2, :, cs] = jnp.clip(bl, 0.0, 255.0) * inv255

    return body


def _diffjpeg(y, cb, cr, quantization, height, width, th):
    B = y.shape[0]
    ny, nc = y.shape[1], cb.shape[1]
    assert ny == (height // 8) * (width // 8) and nc == (height // 16) * (width // 16)
    assert width % 128 == 0

    y_t, c_t = _jpeg_quant_tables()
    b64 = _idct_basis()
    gy = _pack_basis(y_t.reshape(64, 1) * b64, pack=16, dup=1)
    gc = _pack_basis(c_t.reshape(64, 1) * b64, pack=8, dup=2)
    gyh = np.asarray(gy.astype(jnp.bfloat16), np.float32)
    gch = np.asarray(gc.astype(jnp.bfloat16), np.float32)

    yt = jnp.transpose(y, (0, 2, 3, 1)).reshape(B, 64, ny)
    cbt = jnp.transpose(cb, (0, 2, 3, 1)).reshape(B, 64, nc)
    crt = jnp.transpose(cr, (0, 2, 3, 1)).reshape(B, 64, nc)
    rdup = np.concatenate([np.eye(64, dtype=np.float32)] * 2, axis=1)

    tbh, cbh, nxt = th // 8, th // 16, width // 128
    ty, tc = tbh * nxt, cbh * nxt
    tny, tnc = (th // 8) * (width // 8), (th // 16) * (width // 16)

    return pl.pallas_call(
        _fused_kernel(th, width, tny, tnc),
        out_shape=jax.ShapeDtypeStruct((B, 3, height, width), jnp.float32),
        grid=(B, height // th),
        in_specs=[
            pl.BlockSpec(memory_space=pltpu.SMEM),
            pl.BlockSpec((1, 64, tny), lambda b, t: (b, 0, t)),
            pl.BlockSpec((1, 64, tnc), lambda b, t: (b, 0, t)),
            pl.BlockSpec((1, 64, tnc), lambda b, t: (b, 0, t)),
            pl.BlockSpec((64, 128), lambda b, t: (0, 0)),
            pl.BlockSpec((1024, 1024), lambda b, t: (0, 0)),
            pl.BlockSpec((512, 1024), lambda b, t: (0, 0)),
        ],
        out_specs=pl.BlockSpec((1, 3, th, width), lambda b, t: (b, 0, t, 0)),
        scratch_shapes=[
            pltpu.VMEM((tny, 128), jnp.float32),
            pltpu.VMEM((tny // 2, 128), jnp.float32),
            pltpu.VMEM((8 * (ty + 8), 128), jnp.float32),
            pltpu.VMEM((16 * (tc + 8), 128), jnp.float32),
            pltpu.VMEM((16 * (tc + 8), 128), jnp.float32),
        ],
        compiler_params=pltpu.CompilerParams(
            dimension_semantics=("parallel", "parallel"),
            vmem_limit_bytes=100 * 1024 * 1024),
    )(jnp.asarray(quantization, jnp.float32), yt, cbt, crt, jnp.asarray(rdup),
      jnp.asarray(gyh), jnp.asarray(gch))


def kernel(y, cb, cr, quantization):
    return _diffjpeg(y, cb, cr, quantization, 512, 512, 512)


# speedup vs baseline: 1.1299x; 1.0007x over previous
import math
import numpy as np
import jax
import jax.numpy as jnp
from jax.experimental import pallas as pl
from jax.experimental.pallas import tpu as pltpu

_DEFAULT = jax.lax.Precision.DEFAULT


def _jpeg_quant_tables():
    y_table = np.array(
        [[16, 11, 10, 16, 24, 40, 51, 61],
         [12, 12, 14, 19, 26, 58, 60, 55],
         [14, 13, 16, 24, 40, 57, 69, 56],
         [14, 17, 22, 29, 51, 87, 80, 62],
         [18, 22, 37, 56, 68, 109, 103, 77],
         [24, 35, 55, 64, 81, 104, 113, 92],
         [49, 64, 78, 87, 103, 121, 120, 101],
         [72, 92, 95, 98, 112, 100, 103, 99]], dtype=np.float32).T
    c_table = np.full((8, 8), 99.0, dtype=np.float32)
    c_table[:4, :4] = np.array([[17, 18, 24, 47],
                                [18, 21, 26, 66],
                                [24, 26, 56, 99],
                                [47, 66, 99, 99]], dtype=np.float32).T
    return y_table, c_table


def _idct_basis():
    alpha = np.array([1.0 / np.sqrt(2.0)] + [1.0] * 7, dtype=np.float32)
    alpha2 = np.outer(alpha, alpha).astype(np.float32)
    basis = np.zeros((8, 8, 8, 8), dtype=np.float32)
    for x in range(8):
        for y in range(8):
            for u in range(8):
                for v in range(8):
                    basis[x, y, u, v] = (math.cos((2 * u + 1) * x * math.pi / 16) *
                                         math.cos((2 * v + 1) * y * math.pi / 16))
    return (alpha2[:, :, None, None] * basis).reshape(64, 64).astype(np.float32)


def _pack_basis(scaled, pack, dup):
    k = 64 * pack
    g = np.zeros((8, k, 128), np.float32)
    for s1 in range(8):
        cols = np.repeat(scaled[:, s1 * 8:(s1 + 1) * 8], dup, axis=1)
        w = 8 * dup
        for j in range(pack):
            g[s1, j * 64:(j + 1) * 64, j * w:(j + 1) * w] = cols
    return g.transpose(1, 0, 2).reshape(k, 8 * 128)


def _fused_kernel(th, w, tny, tnc):
    tbh = th // 8
    cbh = th // 16
    nxt = w // 128
    ty, tc = tbh * nxt, cbh * nxt
    py, pc = ty + 8, tc + 8

    def body(q_ref, yt_ref, cbt_ref, crt_ref, r_ref, gyh_ref,
             gch_ref, out_ref, dsc_ref, psc_ref,
             ysc_ref, cbsc_ref, crsc_ref):
        b = pl.program_id(0)
        s = q_ref[b] * 0.25
        r = r_ref[...]

        def packed_lhs(t_ref, n, npack):
            dup = jax.lax.dot_general(
                t_ref[0], r, (((0,), (0,)), ((), ())),
                preferred_element_type=jnp.float32, precision=_DEFAULT)
            dsc_ref[0:n, :] = dup
            ev = dsc_ref[pl.ds(0, n // 2, 2)]
            od = dsc_ref[pl.ds(1, n // 2, 2)]
            lane = jax.lax.broadcasted_iota(jnp.int32, (n // 2, 128), 1)
            psc_ref[0:n // 2, :] = jnp.where(lane < 64, ev, od)
            half = npack // 2
            return jnp.concatenate(
                [psc_ref[pl.ds(jp, n // npack, half)] for jp in range(half)],
                axis=1)

        ylhs = packed_lhs(yt_ref, tny, 16)
        ymm = jnp.dot(ylhs, gyh_ref[...], preferred_element_type=jnp.float32,
                      precision=_DEFAULT)
        ymm = ymm * s + 128.0
        for s1 in range(8):
            ysc_ref[s1 * py:s1 * py + ty, :] = ymm[:, s1 * 128:(s1 + 1) * 128]

        cblhs = packed_lhs(cbt_ref, tnc, 8)
        crlhs = packed_lhs(crt_ref, tnc, 8)
        cbf = jnp.concatenate([cblhs, crlhs], axis=0)
        cmm = jnp.dot(cbf, gch_ref[...], preferred_element_type=jnp.float32,
                      precision=_DEFAULT)
        cmm = cmm * s
        for s1 in range(8):
            cbp = cmm[:tc, s1 * 128:(s1 + 1) * 128]
            crp = cmm[tc:, s1 * 128:(s1 + 1) * 128]
            for e in (0, 1):
                yp = (2 * s1 + e) * pc
                cbsc_ref[yp:yp + tc, :] = cbp
                crsc_ref[yp:yp + tc, :] = crp

        inv255 = 1.0 / 255.0
        for xt in range(nxt):
            yb = jnp.concatenate(
                [ysc_ref[pl.ds(br * nxt + xt, 8, py)] for br in range(tbh)],
                axis=0)
            cbb = jnp.concatenate(
                [cbsc_ref[pl.ds((g % 2) * 8 * pc + (g // 2) * nxt + xt, 8, pc)]
                 for g in range(th // 8)], axis=0)
            crb = jnp.concatenate(
                [crsc_ref[pl.ds((g % 2) * 8 * pc + (g // 2) * nxt + xt, 8, pc)]
                 for g in range(th // 8)], axis=0)
            r_ = yb + 1.402 * crb
            g_ = yb - 0.344136 * cbb - 0.714136 * crb
            bl = yb + 1.772 * cbb
            cs = slice(xt * 128, (xt + 1) * 128)
            out_ref[0, 0, :, cs] = jnp.clip(r_, 0.0, 255.0) * inv255
            out_ref[0, 1, :, cs] = jnp.clip(g_, 0.0, 255.0) * inv255
            out_ref[0, 2, :, cs] = jnp.clip(bl, 0.0, 255.0) * inv255

    return body


def _diffjpeg(y, cb, cr, quantization, height, width, th):
    B = y.shape[0]
    ny, nc = y.shape[1], cb.shape[1]
    assert ny == (height // 8) * (width // 8) and nc == (height // 16) * (width // 16)
    assert width % 128 == 0

    y_t, c_t = _jpeg_quant_tables()
    b64 = _idct_basis()
    gy = _pack_basis(y_t.reshape(64, 1) * b64, pack=16, dup=1)
    gc = _pack_basis(c_t.reshape(64, 1) * b64, pack=8, dup=2)
    gyh = np.asarray(gy.astype(jnp.bfloat16), np.float32)
    gch = np.asarray(gc.astype(jnp.bfloat16), np.float32)

    yt = jnp.transpose(y, (0, 2, 3, 1)).reshape(B, 64, ny)
    cbt = jnp.transpose(cb, (0, 2, 3, 1)).reshape(B, 64, nc)
    crt = jnp.transpose(cr, (0, 2, 3, 1)).reshape(B, 64, nc)
    rdup = np.concatenate([np.eye(64, dtype=np.float32)] * 2, axis=1)

    tbh, cbh, nxt = th // 8, th // 16, width // 128
    ty, tc = tbh * nxt, cbh * nxt
    tny, tnc = (th // 8) * (width // 8), (th // 16) * (width // 16)

    return pl.pallas_call(
        _fused_kernel(th, width, tny, tnc),
        out_shape=jax.ShapeDtypeStruct((B, 3, height, width), jnp.float32),
        grid=(B, height // th),
        in_specs=[
            pl.BlockSpec(memory_space=pltpu.SMEM),
            pl.BlockSpec((1, 64, tny), lambda b, t: (b, 0, t)),
            pl.BlockSpec((1, 64, tnc), lambda b, t: (b, 0, t)),
            pl.BlockSpec((1, 64, tnc), lambda b, t: (b, 0, t)),
            pl.BlockSpec((64, 128), lambda b, t: (0, 0)),
            pl.BlockSpec((1024, 1024), lambda b, t: (0, 0)),
            pl.BlockSpec((512, 1024), lambda b, t: (0, 0)),
        ],
        out_specs=pl.BlockSpec((1, 3, th, width), lambda b, t: (b, 0, t, 0)),
        scratch_shapes=[
            pltpu.VMEM((tny, 128), jnp.float32),
            pltpu.VMEM((tny // 2, 128), jnp.float32),
            pltpu.VMEM((8 * (ty + 8), 128), jnp.float32),
            pltpu.VMEM((16 * (tc + 8), 128), jnp.float32),
            pltpu.VMEM((16 * (tc + 8), 128), jnp.float32),
        ],
        compiler_params=pltpu.CompilerParams(
            dimension_semantics=("arbitrary", "arbitrary"),
            vmem_limit_bytes=100 * 1024 * 1024),
    )(jnp.asarray(quantization, jnp.float32), yt, cbt, crt, jnp.asarray(rdup),
      jnp.asarray(gyh), jnp.asarray(gch))


def kernel(y, cb, cr, quantization):
    return _diffjpeg(y, cb, cr, quantization, 512, 512, 512)


# speedup vs baseline: 1.1323x; 1.0022x over previous
import math
import numpy as np
import jax
import jax.numpy as jnp
from jax.experimental import pallas as pl
from jax.experimental.pallas import tpu as pltpu

_DEFAULT = jax.lax.Precision.DEFAULT


def _jpeg_quant_tables():
    y_table = np.array(
        [[16, 11, 10, 16, 24, 40, 51, 61],
         [12, 12, 14, 19, 26, 58, 60, 55],
         [14, 13, 16, 24, 40, 57, 69, 56],
         [14, 17, 22, 29, 51, 87, 80, 62],
         [18, 22, 37, 56, 68, 109, 103, 77],
         [24, 35, 55, 64, 81, 104, 113, 92],
         [49, 64, 78, 87, 103, 121, 120, 101],
         [72, 92, 95, 98, 112, 100, 103, 99]], dtype=np.float32).T
    c_table = np.full((8, 8), 99.0, dtype=np.float32)
    c_table[:4, :4] = np.array([[17, 18, 24, 47],
                                [18, 21, 26, 66],
                                [24, 26, 56, 99],
                                [47, 66, 99, 99]], dtype=np.float32).T
    return y_table, c_table


def _idct_basis():
    alpha = np.array([1.0 / np.sqrt(2.0)] + [1.0] * 7, dtype=np.float32)
    alpha2 = np.outer(alpha, alpha).astype(np.float32)
    basis = np.zeros((8, 8, 8, 8), dtype=np.float32)
    for x in range(8):
        for y in range(8):
            for u in range(8):
                for v in range(8):
                    basis[x, y, u, v] = (math.cos((2 * u + 1) * x * math.pi / 16) *
                                         math.cos((2 * v + 1) * y * math.pi / 16))
    return (alpha2[:, :, None, None] * basis).reshape(64, 64).astype(np.float32)


def _pack_basis(scaled, pack, dup):
    k = 64 * pack
    g = np.zeros((8, k, 128), np.float32)
    for s1 in range(8):
        cols = np.repeat(scaled[:, s1 * 8:(s1 + 1) * 8], dup, axis=1)
        w = 8 * dup
        for j in range(pack):
            g[s1, j * 64:(j + 1) * 64, j * w:(j + 1) * w] = cols
    return g.transpose(1, 0, 2).reshape(k, 8 * 128)


def _fused_kernel(th, w, tny, tnc):
    tbh = th // 8
    cbh = th // 16
    nxt = w // 128
    ty, tc = tbh * nxt, cbh * nxt
    py, pc = ty + 8, tc + 8

    def body(q_ref, yt_ref, cbt_ref, crt_ref, r_ref, gyh_ref,
             gch_ref, out_ref, dsc_ref, psc_ref,
             ysc_ref, cbsc_ref, crsc_ref):
        b = pl.program_id(0)
        s = q_ref[b] * 0.25
        r = r_ref[...]

        def packed_lhs(t_ref, n, npack):
            dup = jax.lax.dot_general(
                t_ref[0], r, (((0,), (0,)), ((), ())),
                preferred_element_type=jnp.float32, precision=_DEFAULT)
            dsc_ref[0:n, :] = dup
            ev = dsc_ref[pl.ds(0, n // 2, 2)]
            od = dsc_ref[pl.ds(1, n // 2, 2)]
            lane = jax.lax.broadcasted_iota(jnp.int32, (n // 2, 128), 1)
            psc_ref[0:n // 2, :] = jnp.where(lane < 64, ev, od)
            half = npack // 2
            return jnp.concatenate(
                [psc_ref[pl.ds(jp, n // npack, half)] for jp in range(half)],
                axis=1)

        ylhs = packed_lhs(yt_ref, tny, 16)
        ymm = jnp.dot(ylhs, gyh_ref[...], preferred_element_type=jnp.float32,
                      precision=_DEFAULT)
        ymm = ymm * s + 128.0
        for s1 in range(8):
            ysc_ref[s1 * py:s1 * py + ty, :] = ymm[:, s1 * 128:(s1 + 1) * 128]

        cblhs = packed_lhs(cbt_ref, tnc, 8)
        crlhs = packed_lhs(crt_ref, tnc, 8)
        cbf = jnp.concatenate([cblhs, crlhs], axis=0)
        cmm = jnp.dot(cbf, gch_ref[...], preferred_element_type=jnp.float32,
                      precision=_DEFAULT)
        cmm = cmm * s
        for s1 in range(8):
            cbp = cmm[:tc, s1 * 128:(s1 + 1) * 128]
            crp = cmm[tc:, s1 * 128:(s1 + 1) * 128]
            for e in (0, 1):
                yp = (2 * s1 + e) * pc
                cbsc_ref[yp:yp + tc, :] = cbp
                crsc_ref[yp:yp + tc, :] = crp

        inv255 = 1.0 / 255.0
        for xt in range(nxt):
            yb = jnp.concatenate(
                [ysc_ref[pl.ds(br * nxt + xt, 8, py)] for br in range(tbh)],
                axis=0)
            cbb = jnp.concatenate(
                [cbsc_ref[pl.ds((g % 2) * 8 * pc + (g // 2) * nxt + xt, 8, pc)]
                 for g in range(th // 8)], axis=0)
            crb = jnp.concatenate(
                [crsc_ref[pl.ds((g % 2) * 8 * pc + (g // 2) * nxt + xt, 8, pc)]
                 for g in range(th // 8)], axis=0)
            r_ = yb + 1.402 * crb
            g_ = yb - 0.344136 * cbb - 0.714136 * crb
            bl = yb + 1.772 * cbb
            cs = slice(xt * 128, (xt + 1) * 128)
            out_ref[0, 0, :, cs] = jnp.clip(r_, 0.0, 255.0) * inv255
            out_ref[0, 1, :, cs] = jnp.clip(g_, 0.0, 255.0) * inv255
            out_ref[0, 2, :, cs] = jnp.clip(bl, 0.0, 255.0) * inv255

    return body


def _diffjpeg(y, cb, cr, quantization, height, width, th):
    B = y.shape[0]
    ny, nc = y.shape[1], cb.shape[1]
    assert ny == (height // 8) * (width // 8) and nc == (height // 16) * (width // 16)
    assert width % 128 == 0

    y_t, c_t = _jpeg_quant_tables()
    b64 = _idct_basis()
    gy = _pack_basis(y_t.reshape(64, 1) * b64, pack=16, dup=1)
    gc = _pack_basis(c_t.reshape(64, 1) * b64, pack=8, dup=2)
    gyh = np.asarray(gy.astype(jnp.bfloat16), np.float32)
    gch = np.asarray(gc.astype(jnp.bfloat16), np.float32)

    yt = jnp.transpose(y, (0, 2, 3, 1)).reshape(B, 64, ny)
    cbt = jnp.transpose(cb, (0, 2, 3, 1)).reshape(B, 64, nc)
    crt = jnp.transpose(cr, (0, 2, 3, 1)).reshape(B, 64, nc)
    rdup = np.concatenate([np.eye(64, dtype=np.float32)] * 2, axis=1)

    tbh, cbh, nxt = th // 8, th // 16, width // 128
    ty, tc = tbh * nxt, cbh * nxt
    tny, tnc = (th // 8) * (width // 8), (th // 16) * (width // 16)

    return pl.pallas_call(
        _fused_kernel(th, width, tny, tnc),
        out_shape=jax.ShapeDtypeStruct((B, 3, height, width), jnp.float32),
        grid=(B, height // th),
        in_specs=[
            pl.BlockSpec(memory_space=pltpu.SMEM),
            pl.BlockSpec((1, 64, tny), lambda b, t: (b, 0, t)),
            pl.BlockSpec((1, 64, tnc), lambda b, t: (b, 0, t)),
            pl.BlockSpec((1, 64, tnc), lambda b, t: (b, 0, t)),
            pl.BlockSpec((64, 128), lambda b, t: (0, 0)),
            pl.BlockSpec((1024, 1024), lambda b, t: (0, 0)),
            pl.BlockSpec((512, 1024), lambda b, t: (0, 0)),
        ],
        out_specs=pl.BlockSpec((1, 3, th, width), lambda b, t: (b, 0, t, 0)),
        scratch_shapes=[
            pltpu.VMEM((tny, 128), jnp.float32),
            pltpu.VMEM((tny // 2, 128), jnp.float32),
            pltpu.VMEM((8 * (ty + 8), 128), jnp.float32),
            pltpu.VMEM((16 * (tc + 8), 128), jnp.float32),
            pltpu.VMEM((16 * (tc + 8), 128), jnp.float32),
        ],
        compiler_params=pltpu.CompilerParams(
            dimension_semantics=("parallel", "parallel"),
            vmem_limit_bytes=100 * 1024 * 1024),
    )(jnp.asarray(quantization, jnp.float32), yt, cbt, crt, jnp.asarray(rdup),
      jnp.asarray(gyh), jnp.asarray(gch))


def kernel(y, cb, cr, quantization):
    return _diffjpeg(y, cb, cr, quantization, 512, 512, 512)
